# baseline (speedup 1.0000x reference)
import functools

import jax
import jax.numpy as jnp
from jax import lax
from jax.experimental import pallas as pl
from jax.experimental.pallas import tpu as pltpu
from jax.experimental.pallas import tpu_sc as plsc

N_NODES = 10000
N_EDGES = 320000
D = 128

NC = 2
NS = 16
NW = NC * NS
CHUNK = 128
E_PER_TILE = 10240
E_PAD = NW * E_PER_TILE
N_ACC = 10112
ROWS_PER_TILE = N_ACC // NS
N_CHUNKS = E_PER_TILE // CHUNK
NBUF = 2

N0_CHUNKS = 128
N1_CHUNKS = 32
NMAX_CHUNKS = max(N0_CHUNKS, N1_CHUNKS)

_vector_mesh = plsc.VectorSubcoreMesh(
    core_axis_name="c", subcore_axis_name="s", num_cores=NC, num_subcores=NS
)


@functools.partial(
    pl.kernel,
    out_type=jax.ShapeDtypeStruct((NC, N_ACC, D), jnp.float32),
    mesh=_vector_mesh,
    scratch_types=[
        pltpu.VMEM_SHARED((N_ACC, D), jnp.float32),
        pltpu.VMEM((N_CHUNKS, CHUNK), jnp.int32),
        pltpu.VMEM((CHUNK, D), jnp.float32),
        pltpu.SemaphoreType.DMA,
    ],
)
def _sc_degree(dst_hbm, ones_hbm, zeros_hbm, out_hbm, acc_sp, idx_v, ones_v, sem):
    c = lax.axis_index("c")
    s = lax.axis_index("s")
    wid = c * NS + s
    z0 = s * ROWS_PER_TILE
    pltpu.sync_copy(zeros_hbm.at[pl.ds(z0, ROWS_PER_TILE)],
                    acc_sp.at[pl.ds(z0, ROWS_PER_TILE)])
    pltpu.sync_copy(ones_hbm, ones_v)
    pltpu.sync_copy(dst_hbm.at[wid], idx_v)
    plsc.subcore_barrier()

    @pl.loop(0, N_CHUNKS)
    def _(j):
        pltpu.sync_copy(ones_v, acc_sp.at[idx_v.at[j]], add=True)

    plsc.subcore_barrier()
    pltpu.sync_copy(acc_sp.at[pl.ds(z0, ROWS_PER_TILE)],
                    out_hbm.at[c, pl.ds(z0, ROWS_PER_TILE)])


@functools.partial(
    pl.kernel,
    out_type=jax.ShapeDtypeStruct((NC, N_ACC, D), jnp.float32),
    mesh=_vector_mesh,
    scratch_types=[
        pltpu.VMEM_SHARED((N_ACC, D), jnp.float32),
        pltpu.VMEM((NBUF, CHUNK), jnp.int32),
        pltpu.VMEM((NMAX_CHUNKS, CHUNK), jnp.int32),
        pltpu.VMEM((NBUF, CHUNK, D), jnp.float32),
        pltpu.SemaphoreType.DMA,
        pltpu.SemaphoreType.DMA,
        pltpu.SemaphoreType.DMA,
        pltpu.SemaphoreType.DMA,
    ],
)
def _sc_edge_pass(y_hbm, src_hbm, dst_hbm, zeros_hbm, out_hbm,
                  acc_sp, src_v, dst_v, rows_v, s0, s1, s2, s3):
    c = lax.axis_index("c")
    s = lax.axis_index("s")
    wid = c * NS + s
    z0 = s * ROWS_PER_TILE
    pltpu.sync_copy(zeros_hbm.at[pl.ds(z0, ROWS_PER_TILE)],
                    acc_sp.at[pl.ds(z0, ROWS_PER_TILE)])
    pltpu.sync_copy(dst_hbm.at[wid], dst_v)
    plsc.subcore_barrier()

    idx_sems = [s0, s1]
    row_sems = [s2, s3]

    n_iters = lax.select(c == 0, N0_CHUNKS // NBUF, N1_CHUNKS // NBUF)

    @pl.loop(0, n_iters)
    def _(j):
        base = j * NBUF
        idx_cp = [
            pltpu.async_copy(src_hbm.at[wid, base + k], src_v.at[k],
                             idx_sems[k])
            for k in range(NBUF)
        ]
        row_cp = []
        for k in range(NBUF):
            idx_cp[k].wait()
            row_cp.append(
                pltpu.async_copy(y_hbm.at[src_v.at[k]], rows_v.at[k],
                                 row_sems[k]))
        for k in range(NBUF):
            row_cp[k].wait()
            pltpu.sync_copy(rows_v.at[k], acc_sp.at[dst_v.at[base + k]],
                            add=True)

    plsc.subcore_barrier()
    pltpu.sync_copy(acc_sp.at[pl.ds(z0, ROWS_PER_TILE)],
                    out_hbm.at[c, pl.ds(z0, ROWS_PER_TILE)])


_ROWS_BLK = 400
_GRID = N_NODES // _ROWS_BLK


def _tc_matmul_body(x_ref, w_ref, o_ref):
    o_ref[...] = jnp.dot(x_ref[...], w_ref[...],
                         preferred_element_type=jnp.float32)


def _tc_matmul(x, w):
    return pl.pallas_call(
        _tc_matmul_body,
        grid=(_GRID,),
        in_specs=[
            pl.BlockSpec((_ROWS_BLK, D), lambda i: (i, 0)),
            pl.BlockSpec((D, D), lambda i: (0, 0)),
        ],
        out_specs=pl.BlockSpec((_ROWS_BLK, D), lambda i: (i, 0)),
        out_shape=jax.ShapeDtypeStruct((N_NODES, D), jnp.float32),
    )(x, w)


def _tc_norm_body(degp_ref, xw_ref, dinv_ref, y_ref):
    deg = degp_ref[0, :, 0] + degp_ref[1, :, 0] + 1.0
    dinv = lax.rsqrt(deg)
    dinv_ref[...] = dinv[:, None]
    y_ref[...] = xw_ref[...] * dinv[:, None]


def _tc_norm(deg_parts, xw):
    return pl.pallas_call(
        _tc_norm_body,
        grid=(_GRID,),
        in_specs=[
            pl.BlockSpec((NC, _ROWS_BLK, D), lambda i: (0, i, 0)),
            pl.BlockSpec((_ROWS_BLK, D), lambda i: (i, 0)),
        ],
        out_specs=[
            pl.BlockSpec((_ROWS_BLK, 1), lambda i: (i, 0)),
            pl.BlockSpec((_ROWS_BLK, D), lambda i: (i, 0)),
        ],
        out_shape=[
            jax.ShapeDtypeStruct((N_NODES, 1), jnp.float32),
            jax.ShapeDtypeStruct((N_NODES, D), jnp.float32),
        ],
    )(deg_parts, xw)


def _tc_mid_body(accp_ref, y1_ref, dinv_ref, w_ref, b_ref, y2_ref):
    acc = accp_ref[0] + accp_ref[1] + y1_ref[...]
    h = jnp.maximum(acc * dinv_ref[...] + b_ref[...], 0.0)
    y2_ref[...] = jnp.dot(h, w_ref[...],
                          preferred_element_type=jnp.float32) * dinv_ref[...]


def _tc_mid(acc_parts, y1, dinv, w2, b1):
    return pl.pallas_call(
        _tc_mid_body,
        grid=(_GRID,),
        in_specs=[
            pl.BlockSpec((NC, _ROWS_BLK, D), lambda i: (0, i, 0)),
            pl.BlockSpec((_ROWS_BLK, D), lambda i: (i, 0)),
            pl.BlockSpec((_ROWS_BLK, 1), lambda i: (i, 0)),
            pl.BlockSpec((D, D), lambda i: (0, 0)),
            pl.BlockSpec((1, D), lambda i: (0, 0)),
        ],
        out_specs=pl.BlockSpec((_ROWS_BLK, D), lambda i: (i, 0)),
        out_shape=jax.ShapeDtypeStruct((N_NODES, D), jnp.float32),
    )(acc_parts, y1, dinv, w2, b1)


def _tc_final_body(accp_ref, y2_ref, dinv_ref, b_ref, o_ref):
    acc = accp_ref[0] + accp_ref[1] + y2_ref[...]
    o_ref[...] = acc * dinv_ref[...] + b_ref[...]


def _tc_final(acc_parts, y2, dinv, b2):
    return pl.pallas_call(
        _tc_final_body,
        grid=(_GRID,),
        in_specs=[
            pl.BlockSpec((NC, _ROWS_BLK, D), lambda i: (0, i, 0)),
            pl.BlockSpec((_ROWS_BLK, D), lambda i: (i, 0)),
            pl.BlockSpec((_ROWS_BLK, 1), lambda i: (i, 0)),
            pl.BlockSpec((1, D), lambda i: (0, 0)),
        ],
        out_specs=pl.BlockSpec((_ROWS_BLK, D), lambda i: (i, 0)),
        out_shape=jax.ShapeDtypeStruct((N_NODES, D), jnp.float32),
    )(acc_parts, y2, dinv, b2)


@jax.jit
def _run(x, edge_index, W1, b1, W2, b2):
    pad = E_PAD - N_EDGES
    src = jnp.concatenate(
        [edge_index[0].astype(jnp.int32), jnp.zeros((pad,), jnp.int32)])
    dst = jnp.concatenate(
        [edge_index[1].astype(jnp.int32),
         N_NODES + (jnp.arange(pad, dtype=jnp.int32) % (N_ACC - N_NODES))])

    def _split(flat):
        ch = flat.reshape(-1, CHUNK)
        c0 = ch[:NS * N0_CHUNKS].reshape(NS, N0_CHUNKS, CHUNK)
        c1 = ch[NS * N0_CHUNKS:].reshape(NS, N1_CHUNKS, CHUNK)
        c0 = jnp.pad(c0, ((0, 0), (0, NMAX_CHUNKS - N0_CHUNKS), (0, 0)))
        c1 = jnp.pad(c1, ((0, 0), (0, NMAX_CHUNKS - N1_CHUNKS), (0, 0)))
        return jnp.concatenate([c0, c1], axis=0)

    src_t = _split(src)
    dst_t = _split(dst)
    dst32 = dst.reshape(NW, N_CHUNKS, CHUNK)
    onesD = jnp.ones((CHUNK, D), jnp.float32)
    zerosD = jnp.zeros((N_ACC, D), jnp.float32)
    b1r = b1.reshape(1, D)
    b2r = b2.reshape(1, D)

    deg_parts = _sc_degree(dst32, onesD, zerosD)
    xw1 = _tc_matmul(x, W1)
    dinv, y1 = _tc_norm(deg_parts[:, :N_NODES], xw1)
    acc1 = _sc_edge_pass(y1, src_t, dst_t, zerosD)
    y2 = _tc_mid(acc1[:, :N_NODES], y1, dinv, W2, b1r)
    acc2 = _sc_edge_pass(y2, src_t, dst_t, zerosD)
    return _tc_final(acc2[:, :N_NODES], y2, dinv, b2r)


def kernel(x, edge_index, W1, b1, W2, b2):
    return _run(x, edge_index, W1, b1, W2, b2)


# speedup vs baseline: 1.0243x; 1.0243x over previous
import functools

import jax
import jax.numpy as jnp
from jax import lax
from jax.experimental import pallas as pl
from jax.experimental.pallas import tpu as pltpu
from jax.experimental.pallas import tpu_sc as plsc

N_NODES = 10000
N_EDGES = 320000
D = 128

NC = 2
NS = 16
NW = NC * NS
CHUNK = 128
E_PER_TILE = 10240
E_PAD = NW * E_PER_TILE
N_ACC = 10112
ROWS_PER_TILE = N_ACC // NS
N_CHUNKS = E_PER_TILE // CHUNK
NBUF = 2

N0_CHUNKS = 124
N1_CHUNKS = 36
NMAX_CHUNKS = max(N0_CHUNKS, N1_CHUNKS)

_vector_mesh = plsc.VectorSubcoreMesh(
    core_axis_name="c", subcore_axis_name="s", num_cores=NC, num_subcores=NS
)


@functools.partial(
    pl.kernel,
    out_type=jax.ShapeDtypeStruct((NC, N_ACC, D), jnp.float32),
    mesh=_vector_mesh,
    scratch_types=[
        pltpu.VMEM_SHARED((N_ACC, D), jnp.float32),
        pltpu.VMEM((N_CHUNKS, CHUNK), jnp.int32),
        pltpu.VMEM((CHUNK, D), jnp.float32),
        pltpu.SemaphoreType.DMA,
    ],
)
def _sc_degree(dst_hbm, ones_hbm, zeros_hbm, out_hbm, acc_sp, idx_v, ones_v, sem):
    c = lax.axis_index("c")
    s = lax.axis_index("s")
    wid = c * NS + s
    z0 = s * ROWS_PER_TILE
    pltpu.sync_copy(zeros_hbm.at[pl.ds(z0, ROWS_PER_TILE)],
                    acc_sp.at[pl.ds(z0, ROWS_PER_TILE)])
    pltpu.sync_copy(ones_hbm, ones_v)
    pltpu.sync_copy(dst_hbm.at[wid], idx_v)
    plsc.subcore_barrier()

    @pl.loop(0, N_CHUNKS)
    def _(j):
        pltpu.sync_copy(ones_v, acc_sp.at[idx_v.at[j]], add=True)

    plsc.subcore_barrier()
    pltpu.sync_copy(acc_sp.at[pl.ds(z0, ROWS_PER_TILE)],
                    out_hbm.at[c, pl.ds(z0, ROWS_PER_TILE)])


@functools.partial(
    pl.kernel,
    out_type=jax.ShapeDtypeStruct((NC, N_ACC, D), jnp.float32),
    mesh=_vector_mesh,
    scratch_types=[
        pltpu.VMEM_SHARED((N_ACC, D), jnp.float32),
        pltpu.VMEM((NBUF, CHUNK), jnp.int32),
        pltpu.VMEM((NMAX_CHUNKS, CHUNK), jnp.int32),
        pltpu.VMEM((NBUF, CHUNK, D), jnp.float32),
        pltpu.SemaphoreType.DMA,
        pltpu.SemaphoreType.DMA,
        pltpu.SemaphoreType.DMA,
        pltpu.SemaphoreType.DMA,
    ],
)
def _sc_edge_pass(y_hbm, src_hbm, dst_hbm, zeros_hbm, out_hbm,
                  acc_sp, src_v, dst_v, rows_v, s0, s1, s2, s3):
    c = lax.axis_index("c")
    s = lax.axis_index("s")
    wid = c * NS + s
    z0 = s * ROWS_PER_TILE
    pltpu.sync_copy(zeros_hbm.at[pl.ds(z0, ROWS_PER_TILE)],
                    acc_sp.at[pl.ds(z0, ROWS_PER_TILE)])
    pltpu.sync_copy(dst_hbm.at[wid], dst_v)
    plsc.subcore_barrier()

    idx_sems = [s0, s1]
    row_sems = [s2, s3]

    n_iters = lax.select(c == 0, N0_CHUNKS // NBUF, N1_CHUNKS // NBUF)

    @pl.loop(0, n_iters)
    def _(j):
        base = j * NBUF
        idx_cp = [
            pltpu.async_copy(src_hbm.at[wid, base + k], src_v.at[k],
                             idx_sems[k])
            for k in range(NBUF)
        ]
        row_cp = []
        for k in range(NBUF):
            idx_cp[k].wait()
            row_cp.append(
                pltpu.async_copy(y_hbm.at[src_v.at[k]], rows_v.at[k],
                                 row_sems[k]))
        for k in range(NBUF):
            row_cp[k].wait()
            pltpu.sync_copy(rows_v.at[k], acc_sp.at[dst_v.at[base + k]],
                            add=True)

    plsc.subcore_barrier()
    pltpu.sync_copy(acc_sp.at[pl.ds(z0, ROWS_PER_TILE)],
                    out_hbm.at[c, pl.ds(z0, ROWS_PER_TILE)])


_ROWS_BLK = 400
_GRID = N_NODES // _ROWS_BLK


def _tc_matmul_body(x_ref, w_ref, o_ref):
    o_ref[...] = jnp.dot(x_ref[...], w_ref[...],
                         preferred_element_type=jnp.float32)


def _tc_matmul(x, w):
    return pl.pallas_call(
        _tc_matmul_body,
        grid=(_GRID,),
        in_specs=[
            pl.BlockSpec((_ROWS_BLK, D), lambda i: (i, 0)),
            pl.BlockSpec((D, D), lambda i: (0, 0)),
        ],
        out_specs=pl.BlockSpec((_ROWS_BLK, D), lambda i: (i, 0)),
        out_shape=jax.ShapeDtypeStruct((N_NODES, D), jnp.float32),
    )(x, w)


def _tc_norm_body(degp_ref, xw_ref, dinv_ref, y_ref):
    deg = degp_ref[0, :, 0] + degp_ref[1, :, 0] + 1.0
    dinv = lax.rsqrt(deg)
    dinv_ref[...] = dinv[:, None]
    y_ref[...] = xw_ref[...] * dinv[:, None]


def _tc_norm(deg_parts, xw):
    return pl.pallas_call(
        _tc_norm_body,
        grid=(_GRID,),
        in_specs=[
            pl.BlockSpec((NC, _ROWS_BLK, D), lambda i: (0, i, 0)),
            pl.BlockSpec((_ROWS_BLK, D), lambda i: (i, 0)),
        ],
        out_specs=[
            pl.BlockSpec((_ROWS_BLK, 1), lambda i: (i, 0)),
            pl.BlockSpec((_ROWS_BLK, D), lambda i: (i, 0)),
        ],
        out_shape=[
            jax.ShapeDtypeStruct((N_NODES, 1), jnp.float32),
            jax.ShapeDtypeStruct((N_NODES, D), jnp.float32),
        ],
    )(deg_parts, xw)


def _tc_mid_body(accp_ref, y1_ref, dinv_ref, w_ref, b_ref, y2_ref):
    acc = accp_ref[0] + accp_ref[1] + y1_ref[...]
    h = jnp.maximum(acc * dinv_ref[...] + b_ref[...], 0.0)
    y2_ref[...] = jnp.dot(h, w_ref[...],
                          preferred_element_type=jnp.float32) * dinv_ref[...]


def _tc_mid(acc_parts, y1, dinv, w2, b1):
    return pl.pallas_call(
        _tc_mid_body,
        grid=(_GRID,),
        in_specs=[
            pl.BlockSpec((NC, _ROWS_BLK, D), lambda i: (0, i, 0)),
            pl.BlockSpec((_ROWS_BLK, D), lambda i: (i, 0)),
            pl.BlockSpec((_ROWS_BLK, 1), lambda i: (i, 0)),
            pl.BlockSpec((D, D), lambda i: (0, 0)),
            pl.BlockSpec((1, D), lambda i: (0, 0)),
        ],
        out_specs=pl.BlockSpec((_ROWS_BLK, D), lambda i: (i, 0)),
        out_shape=jax.ShapeDtypeStruct((N_NODES, D), jnp.float32),
    )(acc_parts, y1, dinv, w2, b1)


def _tc_final_body(accp_ref, y2_ref, dinv_ref, b_ref, o_ref):
    acc = accp_ref[0] + accp_ref[1] + y2_ref[...]
    o_ref[...] = acc * dinv_ref[...] + b_ref[...]


def _tc_final(acc_parts, y2, dinv, b2):
    return pl.pallas_call(
        _tc_final_body,
        grid=(_GRID,),
        in_specs=[
            pl.BlockSpec((NC, _ROWS_BLK, D), lambda i: (0, i, 0)),
            pl.BlockSpec((_ROWS_BLK, D), lambda i: (i, 0)),
            pl.BlockSpec((_ROWS_BLK, 1), lambda i: (i, 0)),
            pl.BlockSpec((1, D), lambda i: (0, 0)),
        ],
        out_specs=pl.BlockSpec((_ROWS_BLK, D), lambda i: (i, 0)),
        out_shape=jax.ShapeDtypeStruct((N_NODES, D), jnp.float32),
    )(acc_parts, y2, dinv, b2)


@jax.jit
def _run(x, edge_index, W1, b1, W2, b2):
    pad = E_PAD - N_EDGES
    src = jnp.concatenate(
        [edge_index[0].astype(jnp.int32), jnp.zeros((pad,), jnp.int32)])
    dst = jnp.concatenate(
        [edge_index[1].astype(jnp.int32),
         N_NODES + (jnp.arange(pad, dtype=jnp.int32) % (N_ACC - N_NODES))])

    def _split(flat):
        ch = flat.reshape(-1, CHUNK)
        c0 = ch[:NS * N0_CHUNKS].reshape(NS, N0_CHUNKS, CHUNK)
        c1 = ch[NS * N0_CHUNKS:].reshape(NS, N1_CHUNKS, CHUNK)
        c0 = jnp.pad(c0, ((0, 0), (0, NMAX_CHUNKS - N0_CHUNKS), (0, 0)))
        c1 = jnp.pad(c1, ((0, 0), (0, NMAX_CHUNKS - N1_CHUNKS), (0, 0)))
        return jnp.concatenate([c0, c1], axis=0)

    src_t = _split(src)
    dst_t = _split(dst)
    dst32 = dst.reshape(NW, N_CHUNKS, CHUNK)
    onesD = jnp.ones((CHUNK, D), jnp.float32)
    zerosD = jnp.zeros((N_ACC, D), jnp.float32)
    b1r = b1.reshape(1, D)
    b2r = b2.reshape(1, D)

    deg_parts = _sc_degree(dst32, onesD, zerosD)
    xw1 = _tc_matmul(x, W1)
    dinv, y1 = _tc_norm(deg_parts[:, :N_NODES], xw1)
    acc1 = _sc_edge_pass(y1, src_t, dst_t, zerosD)
    y2 = _tc_mid(acc1[:, :N_NODES], y1, dinv, W2, b1r)
    acc2 = _sc_edge_pass(y2, src_t, dst_t, zerosD)
    return _tc_final(acc2[:, :N_NODES], y2, dinv, b2r)


def kernel(x, edge_index, W1, b1, W2, b2):
    return _run(x, edge_index, W1, b1, W2, b2)
